# baseline (speedup 1.0000x reference)
import functools

import jax
import jax.numpy as jnp
from jax import lax
from jax.experimental import pallas as pl
from jax.experimental.pallas import tpu as pltpu
from jax.experimental.pallas import tpu_sc as plsc

NC, NS = 2, 16
NW = NC * NS
B = 16384 * 26
D = 128
BPW = B // NW
CH = 128
NCHUNK = BPW // CH
GRP = 2
NG = NCHUNK // GRP


def _sc_body(idx_hbm, table_hbm, out_hbm, idx_v, rows_v, sem_g, sem_w):
    wid = lax.axis_index("s") * NC + lax.axis_index("c")
    pltpu.sync_copy(idx_hbm.at[wid], idx_v)
    chunk_base = wid * NCHUNK

    def fire_gathers(buf, g):
        for bb in range(GRP):
            pltpu.async_copy(
                table_hbm.at[idx_v.at[g * GRP + bb]], rows_v.at[buf, bb], sem_g)

    def wait_gathers():
        for bb in range(GRP):
            pltpu.make_async_copy(
                table_hbm.at[idx_v.at[bb]], rows_v.at[0, bb], sem_g).wait()

    def start_write(buf, g):
        return pltpu.async_copy(
            rows_v.at[buf], out_hbm.at[pl.ds(chunk_base + g * GRP, GRP)], sem_w)

    def wait_write():
        pltpu.make_async_copy(
            rows_v.at[0], out_hbm.at[pl.ds(chunk_base, GRP)], sem_w).wait()

    fire_gathers(0, 0)

    def body(g, carry):
        buf = lax.rem(g, 2)
        wait_gathers()
        pl.when(g > 0)(wait_write)
        start_write(buf, g)

        def prefetch():
            fire_gathers(1 - buf, g + 1)

        pl.when(g < NG - 1)(prefetch)
        return carry

    lax.fori_loop(0, NG, body, 0)
    wait_write()


@jax.jit
def kernel(indices, table):
    idx3 = indices.astype(jnp.int32).reshape(NW, NCHUNK, CH)
    mesh = plsc.VectorSubcoreMesh(core_axis_name="c", subcore_axis_name="s")
    k = functools.partial(
        pl.kernel,
        out_type=jax.ShapeDtypeStruct((B // CH, CH, D), jnp.float32),
        mesh=mesh,
        scratch_types=[
            pltpu.VMEM((NCHUNK, CH), jnp.int32),
            pltpu.VMEM((2, GRP, CH, D), jnp.float32),
            pltpu.SemaphoreType.DMA,
            pltpu.SemaphoreType.DMA,
        ],
    )(_sc_body)
    out = k(idx3, table)
    return out.reshape(16384, 26, D)


# speedup vs baseline: 4.5767x; 4.5767x over previous
import functools

import jax
import jax.numpy as jnp
from jax import lax
from jax.experimental import pallas as pl
from jax.experimental.pallas import tpu as pltpu
from jax.experimental.pallas import tpu_sc as plsc

NC, NS = 2, 16
NW = NC * NS
B = 16384 * 26
D = 128
BPW = B // NW
CH = 128
NCHUNK = BPW // CH
GRP = 2
NG = NCHUNK // GRP


def _sc_body(idx_hbm, table_hbm, out_hbm, idx_v, rows_v, table_sh, sem_g, sem_w):
    sid = lax.axis_index("s")
    wid = sid * NC + lax.axis_index("c")
    pltpu.sync_copy(idx_hbm.at[wid], idx_v)

    def stage_table():
        pltpu.sync_copy(table_hbm, table_sh)

    pl.when(sid == 0)(stage_table)
    plsc.subcore_barrier()
    chunk_base = wid * NCHUNK

    def fire_gathers(buf, g):
        for bb in range(GRP):
            pltpu.async_copy(
                table_sh.at[idx_v.at[g * GRP + bb]], rows_v.at[buf, bb], sem_g)

    def wait_gathers():
        for bb in range(GRP):
            pltpu.make_async_copy(
                table_sh.at[idx_v.at[bb]], rows_v.at[0, bb], sem_g).wait()

    def start_write(buf, g):
        return pltpu.async_copy(
            rows_v.at[buf], out_hbm.at[pl.ds(chunk_base + g * GRP, GRP)], sem_w)

    def wait_write():
        pltpu.make_async_copy(
            rows_v.at[0], out_hbm.at[pl.ds(chunk_base, GRP)], sem_w).wait()

    fire_gathers(0, 0)

    def body(g, carry):
        buf = lax.rem(g, 2)
        wait_gathers()
        pl.when(g > 0)(wait_write)
        start_write(buf, g)

        def prefetch():
            fire_gathers(1 - buf, g + 1)

        pl.when(g < NG - 1)(prefetch)
        return carry

    lax.fori_loop(0, NG, body, 0)
    wait_write()


@jax.jit
def kernel(indices, table):
    idx3 = indices.astype(jnp.int32).reshape(NW, NCHUNK, CH)
    mesh = plsc.VectorSubcoreMesh(core_axis_name="c", subcore_axis_name="s")
    k = functools.partial(
        pl.kernel,
        out_type=jax.ShapeDtypeStruct((B // CH, CH, D), jnp.float32),
        mesh=mesh,
        scratch_types=[
            pltpu.VMEM((NCHUNK, CH), jnp.int32),
            pltpu.VMEM((2, GRP, CH, D), jnp.float32),
            pltpu.VMEM_SHARED((10, D), jnp.float32),
            pltpu.SemaphoreType.DMA,
            pltpu.SemaphoreType.DMA,
        ],
    )(_sc_body)
    out = k(idx3, table)
    return out.reshape(16384, 26, D)


# speedup vs baseline: 8.1359x; 1.7777x over previous
import functools

import jax
import jax.numpy as jnp
from jax import lax
from jax.experimental import pallas as pl
from jax.experimental.pallas import tpu as pltpu
from jax.experimental.pallas import tpu_sc as plsc

NC, NS = 2, 16
NW = NC * NS
BB = 16384
F = 26
FP = 32
D = 128
BPW = BB // NW
NB = 8
NG = BPW // NB


def _sc_body(idx_hbm, table_hbm, out_hbm, idx_v, rows_v, table_sh, sem_g, sem_w):
    sid = lax.axis_index("s")
    wid = sid * NC + lax.axis_index("c")
    b_base = wid * BPW
    pltpu.sync_copy(idx_hbm.at[pl.ds(b_base * FP, BPW * FP)], idx_v)

    def stage_table():
        pltpu.sync_copy(table_hbm, table_sh)

    pl.when(sid == 0)(stage_table)
    plsc.subcore_barrier()

    def fire_gather(buf, g):
        for bb in range(NB):
            pltpu.async_copy(
                table_sh.at[idx_v.at[pl.ds((g * NB + bb) * FP, F)]],
                rows_v.at[buf, bb], sem_g)

    def wait_gather():
        for bb in range(NB):
            pltpu.make_async_copy(
                table_sh.at[idx_v.at[pl.ds(0, F)]], rows_v.at[0, bb], sem_g).wait()

    def start_write(buf, g):
        pltpu.async_copy(
            rows_v.at[buf], out_hbm.at[pl.ds(b_base + g * NB, NB)], sem_w)

    def wait_write():
        pltpu.make_async_copy(
            rows_v.at[0], out_hbm.at[pl.ds(b_base, NB)], sem_w).wait()

    fire_gather(0, 0)

    def body(g, carry):
        buf = lax.rem(g, 2)
        wait_gather()
        pl.when(g > 0)(wait_write)
        start_write(buf, g)

        def prefetch():
            fire_gather(1 - buf, g + 1)

        pl.when(g < NG - 1)(prefetch)
        return carry

    lax.fori_loop(0, NG, body, 0)
    wait_write()


@jax.jit
def kernel(indices, table):
    mesh = plsc.VectorSubcoreMesh(core_axis_name="c", subcore_axis_name="s")
    k = functools.partial(
        pl.kernel,
        out_type=jax.ShapeDtypeStruct((BB, F, D), jnp.float32),
        mesh=mesh,
        scratch_types=[
            pltpu.VMEM((BPW * FP,), jnp.int32),
            pltpu.VMEM((2, NB, F, D), jnp.float32),
            pltpu.VMEM_SHARED((10, D), jnp.float32),
            pltpu.SemaphoreType.DMA,
            pltpu.SemaphoreType.DMA,
        ],
    )(_sc_body)
    idxp = jnp.pad(indices.astype(jnp.int32), ((0, 0), (0, FP - F)))
    return k(idxp.reshape(-1), table)


# speedup vs baseline: 8.1664x; 1.0037x over previous
import functools

import jax
import jax.numpy as jnp
from jax import lax
from jax.experimental import pallas as pl
from jax.experimental.pallas import tpu as pltpu
from jax.experimental.pallas import tpu_sc as plsc

NC, NS = 2, 16
NW = NC * NS
BB = 16384
F = 26
FP = 32
D = 128
BPW = BB // NW
NB = 8
NG = BPW // NB


def _sc_body(idx_hbm, table_hbm, out_hbm, idx_v, rows_v, table_sh, sem_g, sem_w):
    sid = lax.axis_index("s")
    wid = sid * NC + lax.axis_index("c")
    b_base = wid * BPW
    pltpu.sync_copy(idx_hbm.at[pl.ds(b_base * FP, BPW * FP)], idx_v)

    def stage_table():
        pltpu.sync_copy(table_hbm, table_sh)

    pl.when(sid == 0)(stage_table)
    plsc.subcore_barrier()

    def fire_gather(buf, g):
        for bb in range(NB):
            pltpu.async_copy(
                table_sh.at[idx_v.at[pl.ds((g * NB + bb) * FP, F)]],
                rows_v.at[buf, bb], sem_g)

    def wait_gather():
        for bb in range(NB):
            pltpu.make_async_copy(
                table_sh.at[idx_v.at[pl.ds(0, F)]], rows_v.at[0, bb], sem_g).wait()

    def start_write(buf, g):
        pltpu.async_copy(
            rows_v.at[buf], out_hbm.at[pl.ds(b_base + g * NB, NB)], sem_w)

    def wait_write():
        pltpu.make_async_copy(
            rows_v.at[0], out_hbm.at[pl.ds(b_base, NB)], sem_w).wait()

    fire_gather(0, 0)

    def body(g, carry):
        buf = lax.rem(g, 2)
        wait_gather()
        pl.when(g > 0)(wait_write)
        start_write(buf, g)

        def prefetch():
            fire_gather(1 - buf, g + 1)

        pl.when(g < NG - 1)(prefetch)
        return carry

    lax.fori_loop(0, NG, body, 0)
    wait_write()


@jax.jit
def kernel(indices, table):
    mesh = plsc.VectorSubcoreMesh(core_axis_name="c", subcore_axis_name="s")
    k = functools.partial(
        pl.kernel,
        out_type=jax.ShapeDtypeStruct((BB, F, D), jnp.float32),
        mesh=mesh,
        compiler_params=pltpu.CompilerParams(use_tc_tiling_on_sc=True),
        scratch_types=[
            pltpu.VMEM((BPW * FP,), jnp.int32),
            pltpu.VMEM((2, NB, F, D), jnp.float32),
            pltpu.VMEM_SHARED((10, D), jnp.float32),
            pltpu.SemaphoreType.DMA,
            pltpu.SemaphoreType.DMA,
        ],
    )(_sc_body)
    idxp = jnp.pad(indices.astype(jnp.int32), ((0, 0), (0, FP - F)))
    return k(idxp.reshape(-1), table)


# speedup vs baseline: 22.5855x; 2.7657x over previous
import functools

import jax
import jax.numpy as jnp
from jax import lax
from jax.experimental import pallas as pl
from jax.experimental.pallas import tpu as pltpu
from jax.experimental.pallas import tpu_sc as plsc

NC, NS = 2, 16
NW = NC * NS
BB = 16384
F = 26
D = 128
BPW = BB // NW
CHB = 256
NH = BPW // CHB
CH = 128
NGPB = CHB // CH
NG = F * NH


def _sc_body(idx_hbm, table_hbm, out_hbm, idx_v, rows_v, table_sh, sem_g, sem_w):
    sid = lax.axis_index("s")
    wid = sid * NC + lax.axis_index("c")
    b0 = wid * BPW
    pltpu.sync_copy(idx_hbm.at[:, pl.ds(b0, BPW)], idx_v)

    def stage_table():
        pltpu.sync_copy(table_hbm, table_sh)

    pl.when(sid == 0)(stage_table)
    plsc.subcore_barrier()

    def fire_gather(buf, g):
        f = lax.div(g, NH)
        half = lax.rem(g, NH)
        for k in range(NGPB):
            pltpu.async_copy(
                table_sh.at[idx_v.at[f, pl.ds(half * CHB + k * CH, CH)]],
                rows_v.at[buf, pl.ds(k * CH, CH)], sem_g)

    def wait_gather():
        for k in range(NGPB):
            pltpu.make_async_copy(
                table_sh.at[idx_v.at[0, pl.ds(0, CH)]],
                rows_v.at[0, pl.ds(0, CH)], sem_g).wait()

    def start_write(buf, g):
        f = lax.div(g, NH)
        half = lax.rem(g, NH)
        pltpu.async_copy(
            rows_v.at[buf], out_hbm.at[f, pl.ds(b0 + half * CHB, CHB)], sem_w)

    def wait_write():
        pltpu.make_async_copy(
            rows_v.at[0], out_hbm.at[0, pl.ds(b0, CHB)], sem_w).wait()

    fire_gather(0, 0)

    def body(g, carry):
        buf = lax.rem(g, 2)
        wait_gather()
        pl.when(g > 0)(wait_write)
        start_write(buf, g)

        def prefetch():
            fire_gather(1 - buf, g + 1)

        pl.when(g < NG - 1)(prefetch)
        return carry

    lax.fori_loop(0, NG, body, 0)
    wait_write()


@jax.jit
def kernel(indices, table):
    idx_t = jnp.transpose(indices.astype(jnp.int32))
    mesh = plsc.VectorSubcoreMesh(core_axis_name="c", subcore_axis_name="s")
    k = functools.partial(
        pl.kernel,
        out_type=jax.ShapeDtypeStruct((F, BB, D), jnp.float32),
        mesh=mesh,
        scratch_types=[
            pltpu.VMEM((F, BPW), jnp.int32),
            pltpu.VMEM((2, CHB, D), jnp.float32),
            pltpu.VMEM_SHARED((10, D), jnp.float32),
            pltpu.SemaphoreType.DMA,
            pltpu.SemaphoreType.DMA,
        ],
    )(_sc_body)
    out_t = k(idx_t, table)
    return jnp.transpose(out_t, (1, 0, 2))


# speedup vs baseline: 23.5842x; 1.0442x over previous
import functools

import jax
import jax.numpy as jnp
from jax import lax
from jax.experimental import pallas as pl
from jax.experimental.pallas import tpu as pltpu
from jax.experimental.pallas import tpu_sc as plsc

NC, NS = 2, 16
NW = NC * NS
BB = 16384
F = 26
D = 128
BPW = BB // NW
CHB = 256
NH = BPW // CHB
CH = 128
NGPB = CHB // CH
NG = F * NH


def _sc_body(idx_hbm, table_hbm, out_hbm, idx_v, rows_v, table_sh, sem_g, sem_w):
    sid = lax.axis_index("s")
    wid = sid * NC + lax.axis_index("c")
    b0 = wid * BPW
    pltpu.sync_copy(idx_hbm.at[:, pl.ds(b0, BPW)], idx_v)

    def stage_table():
        pltpu.sync_copy(table_hbm, table_sh)

    pl.when(sid == 0)(stage_table)
    plsc.subcore_barrier()

    def fire_gather(buf, g):
        f = lax.div(g, NH)
        half = lax.rem(g, NH)
        for k in range(NGPB):
            pltpu.async_copy(
                table_sh.at[idx_v.at[f, pl.ds(half * CHB + k * CH, CH)]],
                rows_v.at[buf, pl.ds(k * CH, CH)], sem_g)

    def wait_gather():
        for k in range(NGPB):
            pltpu.make_async_copy(
                table_sh.at[idx_v.at[0, pl.ds(0, CH)]],
                rows_v.at[0, pl.ds(0, CH)], sem_g).wait()

    def start_write(buf, g):
        f = lax.div(g, NH)
        half = lax.rem(g, NH)
        pltpu.async_copy(
            rows_v.at[buf], out_hbm.at[f, pl.ds(b0 + half * CHB, CHB)], sem_w)

    def wait_write():
        pltpu.make_async_copy(
            rows_v.at[0], out_hbm.at[0, pl.ds(b0, CHB)], sem_w).wait()

    fire_gather(0, 0)
    fire_gather(1, 1)

    def body(g, carry):
        wait_gather()
        pl.when(g > 0)(wait_write)
        start_write(lax.rem(g, 3), g)

        def prefetch():
            fire_gather(lax.rem(g + 2, 3), g + 2)

        pl.when(g < NG - 2)(prefetch)
        return carry

    lax.fori_loop(0, NG, body, 0)
    wait_write()


@jax.jit
def kernel(indices, table):
    idx_t = jnp.transpose(indices.astype(jnp.int32))
    mesh = plsc.VectorSubcoreMesh(core_axis_name="c", subcore_axis_name="s")
    k = functools.partial(
        pl.kernel,
        out_type=jax.ShapeDtypeStruct((F, BB, D), jnp.float32),
        mesh=mesh,
        scratch_types=[
            pltpu.VMEM((F, BPW), jnp.int32),
            pltpu.VMEM((3, CHB, D), jnp.float32),
            pltpu.VMEM_SHARED((10, D), jnp.float32),
            pltpu.SemaphoreType.DMA,
            pltpu.SemaphoreType.DMA,
        ],
    )(_sc_body)
    out_t = k(idx_t, table)
    return jnp.transpose(out_t, (1, 0, 2))
